# Initial kernel scaffold; baseline (speedup 1.0000x reference)
import jax
import jax.numpy as jnp
from jax.experimental import pallas as pl


def kernel(x, in_proj_w, in_proj_b, out_proj_w, out_proj_b, ln1_g, ln1_b, ln2_g, ln2_b, gate_w, gate_b, w1, b1, w2, b2):
    raise NotImplementedError("write your pallas kernel here")


# speedup vs baseline: 3.8171x; 3.8171x over previous
import functools

import jax
import jax.numpy as jnp
from jax import lax
from jax.experimental import pallas as pl
from jax.experimental.pallas import tpu as pltpu
from jax.experimental.pallas import tpu_sc as plsc

S, B, D, H, HD = 2048, 2, 1024, 4, 256
E, FF = 32, 4096
N = S * B
RT = 512
TT = 128
GT = N // TT + E
NP = GT * TT


def _qkv_body(x_ref, w_ref, b_ref, o_ref):
    o_ref[...] = lax.dot_general(
        x_ref[...], w_ref[...], (((1,), (1,)), ((), ())),
        preferred_element_type=jnp.float32) + b_ref[0]


def _qkv(x2d, wi, bi3):
    return pl.pallas_call(
        _qkv_body,
        grid=(N // RT, 3),
        in_specs=[
            pl.BlockSpec((RT, D), lambda i, j: (i, 0)),
            pl.BlockSpec((D, D), lambda i, j: (j, 0)),
            pl.BlockSpec((1, 1, D), lambda i, j: (j, 0, 0)),
        ],
        out_specs=pl.BlockSpec((RT, D), lambda i, j: (i, j)),
        out_shape=jax.ShapeDtypeStruct((N, 3 * D), jnp.float32),
    )(x2d, wi, bi3)


def _attn_body(q_ref, k_ref, v_ref, o_ref):
    q = q_ref[0]
    k = k_ref[0]
    v = v_ref[0]
    s = lax.dot_general(q, k, (((1,), (1,)), ((), ())),
                        preferred_element_type=jnp.float32) * (1.0 / 16.0)
    m = jnp.max(s, axis=-1, keepdims=True)
    p = jnp.exp(s - m)
    p = p / jnp.sum(p, axis=-1, keepdims=True)
    o_ref[0] = jnp.dot(p, v, preferred_element_type=jnp.float32)


def _attn(q, k, v):
    qt = 512
    return pl.pallas_call(
        _attn_body,
        grid=(B * H, S // qt),
        in_specs=[
            pl.BlockSpec((1, qt, HD), lambda h, i: (h, i, 0)),
            pl.BlockSpec((1, S, HD), lambda h, i: (h, 0, 0)),
            pl.BlockSpec((1, S, HD), lambda h, i: (h, 0, 0)),
        ],
        out_specs=pl.BlockSpec((1, qt, HD), lambda h, i: (h, i, 0)),
        out_shape=jax.ShapeDtypeStruct((B * H, S, HD), jnp.float32),
    )(q, k, v)


def _post_attn_body(x_ref, a_ref, wo_ref, bo_ref, g_ref, b_ref, gw_ref,
                    gb_ref, h_ref, oh_ref):
    o = lax.dot_general(a_ref[...], wo_ref[...], (((1,), (1,)), ((), ())),
                        preferred_element_type=jnp.float32) + bo_ref[...]
    s = x_ref[...] + o
    m = jnp.mean(s, axis=-1, keepdims=True)
    c = s - m
    v = jnp.mean(c * c, axis=-1, keepdims=True)
    h = c * lax.rsqrt(v + 1e-5) * g_ref[...] + b_ref[...]
    h_ref[...] = h
    logits = jnp.dot(h, gw_ref[...], preferred_element_type=jnp.float32) \
        + gb_ref[...]
    oh = (logits == jnp.max(logits, axis=-1, keepdims=True)) \
        .astype(jnp.float32)
    strict_up = (lax.broadcasted_iota(jnp.int32, (E, E), 0)
                 < lax.broadcasted_iota(jnp.int32, (E, E), 1)) \
        .astype(jnp.float32)
    earlier = jnp.dot(oh, strict_up, preferred_element_type=jnp.float32)
    oh_ref[...] = oh * (earlier == 0.0).astype(jnp.float32)


def _post_attn(x2d, a2d, wo, bo, g1, b1, gw, gb):
    return pl.pallas_call(
        _post_attn_body,
        grid=(N // RT,),
        in_specs=[
            pl.BlockSpec((RT, D), lambda i: (i, 0)),
            pl.BlockSpec((RT, D), lambda i: (i, 0)),
            pl.BlockSpec((D, D), lambda i: (0, 0)),
            pl.BlockSpec((1, D), lambda i: (0, 0)),
            pl.BlockSpec((1, D), lambda i: (0, 0)),
            pl.BlockSpec((1, D), lambda i: (0, 0)),
            pl.BlockSpec((D, E), lambda i: (0, 0)),
            pl.BlockSpec((1, E), lambda i: (0, 0)),
        ],
        out_specs=[
            pl.BlockSpec((RT, D), lambda i: (i, 0)),
            pl.BlockSpec((RT, E), lambda i: (i, 0)),
        ],
        out_shape=[
            jax.ShapeDtypeStruct((N, D), jnp.float32),
            jax.ShapeDtypeStruct((N, E), jnp.float32),
        ],
    )(x2d, a2d, wo, bo, g1, b1, gw, gb)


def _rank_body(oh_ref, rank_ref, cnt_ref, cnt_run):
    t = pl.program_id(0)

    @pl.when(t == 0)
    def _():
        cnt_run[...] = jnp.zeros_like(cnt_run)

    oh = oh_ref[...]
    tril = (lax.broadcasted_iota(jnp.int32, (RT, RT), 1)
            < lax.broadcasted_iota(jnp.int32, (RT, RT), 0)).astype(jnp.float32)
    excl = jnp.dot(tril, oh, preferred_element_type=jnp.float32)
    rank_ref[...] = jnp.sum(oh * (excl + cnt_run[...]), axis=1, keepdims=True)
    cnt_run[...] = cnt_run[...] + jnp.sum(oh, axis=0, keepdims=True)
    cnt_ref[...] = cnt_run[...]


def _ranks(oh):
    return pl.pallas_call(
        _rank_body,
        grid=(N // RT,),
        in_specs=[pl.BlockSpec((RT, E), lambda i: (i, 0))],
        out_specs=[
            pl.BlockSpec((RT, 1), lambda i: (i, 0)),
            pl.BlockSpec((1, E), lambda i: (0, 0)),
        ],
        out_shape=[
            jax.ShapeDtypeStruct((N, 1), jnp.float32),
            jax.ShapeDtypeStruct((1, E), jnp.float32),
        ],
        scratch_shapes=[pltpu.VMEM((1, E), jnp.float32)],
    )(oh)


def _sched_body(cnt_ref, oh_ref, rank_ref, dest_ref, te_ref, vd_ref):
    cnt = cnt_ref[...]
    ntiles = jnp.floor((cnt + (TT - 1)) * (1.0 / TT))
    strict_lo = (lax.broadcasted_iota(jnp.int32, (E, E), 0)
                 < lax.broadcasted_iota(jnp.int32, (E, E), 1)) \
        .astype(jnp.float32)
    start = jnp.dot(ntiles, strict_lo,
                    preferred_element_type=jnp.float32)
    pad_off = start * float(TT)
    oh = oh_ref[...]
    dest = rank_ref[...] + jnp.sum(oh * pad_off, axis=1, keepdims=True)
    dest_ref[...] = dest.astype(jnp.int32)
    total = jnp.sum(ntiles)
    t_iota = lax.broadcasted_iota(jnp.int32, (GT, 1), 0).astype(jnp.float32)
    t_eff = jnp.minimum(t_iota, total - 1.0)
    te = jnp.sum((t_eff >= start).astype(jnp.float32), axis=1,
                 keepdims=True) - 1.0
    te_ref[...] = te.astype(jnp.int32)
    vd_ref[...] = (t_iota < total).astype(jnp.int32)


def _schedule(cnt, oh, rank):
    return pl.pallas_call(
        _sched_body,
        grid=(N // RT,),
        in_specs=[
            pl.BlockSpec((1, E), lambda i: (0, 0)),
            pl.BlockSpec((RT, E), lambda i: (i, 0)),
            pl.BlockSpec((RT, 1), lambda i: (i, 0)),
        ],
        out_specs=[
            pl.BlockSpec((RT, 1), lambda i: (i, 0)),
            pl.BlockSpec((GT, 1), lambda i: (0, 0)),
            pl.BlockSpec((GT, 1), lambda i: (0, 0)),
        ],
        out_shape=[
            jax.ShapeDtypeStruct((N, 1), jnp.int32),
            jax.ShapeDtypeStruct((GT, 1), jnp.int32),
            jax.ShapeDtypeStruct((GT, 1), jnp.int32),
        ],
    )(cnt, oh, rank)


_NW = 32
_CH = 64


def _sc_mesh():
    return plsc.VectorSubcoreMesh(core_axis_name="c", subcore_axis_name="s")


def _sc_scatter(h2d, dest1):
    @functools.partial(
        pl.kernel, mesh=_sc_mesh(),
        out_type=jax.ShapeDtypeStruct((NP, D), jnp.float32),
        scratch_types=[
            pltpu.VMEM((_CH,), jnp.int32),
            pltpu.VMEM((_CH, D), jnp.float32),
            pltpu.SemaphoreType.DMA,
        ],
    )
    def body(h_hbm, dest_hbm, xs_hbm, idx_v, rows_v, sem):
        wid = lax.axis_index("s") * 2 + lax.axis_index("c")
        base = wid * (N // _NW)
        for c in range(N // _NW // _CH):
            off = base + c * _CH
            pltpu.sync_copy(dest_hbm.at[pl.ds(off, _CH)], idx_v)
            pltpu.sync_copy(h_hbm.at[pl.ds(off, _CH)], rows_v)
            pltpu.async_copy(rows_v, xs_hbm.at[idx_v], sem).wait()

    return body(h2d, dest1)


def _ffn_body(te_ref, vd_ref, x_ref, w1_ref, b1_ref, w2_ref, b2_ref, y_ref,
              acc):
    t = pl.program_id(0)
    f = pl.program_id(1)

    @pl.when(vd_ref[t] == 1)
    def _():
        h1 = jax.nn.relu(
            jnp.dot(x_ref[...], w1_ref[0], preferred_element_type=jnp.float32)
            + b1_ref[0, 0])
        part = jnp.dot(h1, w2_ref[0], preferred_element_type=jnp.float32)

        @pl.when(f == 0)
        def _():
            acc[...] = part

        @pl.when(f > 0)
        def _():
            acc[...] = acc[...] + part

        @pl.when(f == FF // D - 1)
        def _():
            y_ref[...] = acc[...] + b2_ref[0]


def _ffn(te, vd, xs, w1, b1, w2, b2):
    nf = FF // D
    grid_spec = pltpu.PrefetchScalarGridSpec(
        num_scalar_prefetch=2,
        grid=(GT, nf),
        in_specs=[
            pl.BlockSpec((TT, D), lambda t, f, te, vd: (t * vd[t], 0)),
            pl.BlockSpec((1, D, D), lambda t, f, te, vd: (te[t], 0, f * vd[t])),
            pl.BlockSpec((1, 1, 1, D),
                         lambda t, f, te, vd: (te[t], f * vd[t], 0, 0)),
            pl.BlockSpec((1, D, D), lambda t, f, te, vd: (te[t], f * vd[t], 0)),
            pl.BlockSpec((1, 1, D), lambda t, f, te, vd: (te[t], 0, 0)),
        ],
        out_specs=pl.BlockSpec((TT, D), lambda t, f, te, vd: (t, 0)),
        scratch_shapes=[pltpu.VMEM((TT, D), jnp.float32)],
    )
    return pl.pallas_call(
        _ffn_body,
        grid_spec=grid_spec,
        out_shape=jax.ShapeDtypeStruct((NP, D), jnp.float32),
    )(te, vd, xs, w1, b1, w2, b2)


def _sc_gather(ys, dest1):
    @functools.partial(
        pl.kernel, mesh=_sc_mesh(),
        out_type=jax.ShapeDtypeStruct((N, D), jnp.float32),
        scratch_types=[
            pltpu.VMEM((_CH,), jnp.int32),
            pltpu.VMEM((_CH, D), jnp.float32),
            pltpu.SemaphoreType.DMA,
        ],
    )
    def body(ys_hbm, dest_hbm, out_hbm, idx_v, rows_v, sem):
        wid = lax.axis_index("s") * 2 + lax.axis_index("c")
        base = wid * (N // _NW)
        for c in range(N // _NW // _CH):
            off = base + c * _CH
            pltpu.sync_copy(dest_hbm.at[pl.ds(off, _CH)], idx_v)
            pltpu.async_copy(ys_hbm.at[idx_v], rows_v, sem).wait()
            pltpu.sync_copy(rows_v, out_hbm.at[pl.ds(off, _CH)])

    return body(ys, dest1)


def _ln2_body(h_ref, y_ref, g_ref, b_ref, o_ref):
    s = h_ref[...] + y_ref[...]
    m = jnp.mean(s, axis=-1, keepdims=True)
    c = s - m
    v = jnp.mean(c * c, axis=-1, keepdims=True)
    o_ref[...] = c * lax.rsqrt(v + 1e-5) * g_ref[...] + b_ref[...]


def _ln2(h2d, ytok, g2, b2):
    return pl.pallas_call(
        _ln2_body,
        grid=(N // RT,),
        in_specs=[
            pl.BlockSpec((RT, D), lambda i: (i, 0)),
            pl.BlockSpec((RT, D), lambda i: (i, 0)),
            pl.BlockSpec((1, D), lambda i: (0, 0)),
            pl.BlockSpec((1, D), lambda i: (0, 0)),
        ],
        out_specs=pl.BlockSpec((RT, D), lambda i: (i, 0)),
        out_shape=jax.ShapeDtypeStruct((N, D), jnp.float32),
    )(h2d, ytok, g2, b2)


def kernel(x, in_proj_w, in_proj_b, out_proj_w, out_proj_b, ln1_g, ln1_b,
           ln2_g, ln2_b, gate_w, gate_b, w1, b1, w2, b2):
    x2d = x.reshape(N, D)
    qkv = _qkv(x2d, in_proj_w, in_proj_b.reshape(3, 1, D))
    qkv_t = qkv.reshape(S, B, 3, H, HD).transpose(2, 1, 3, 0, 4) \
        .reshape(3, B * H, S, HD)
    o = _attn(qkv_t[0], qkv_t[1], qkv_t[2])
    a2d = o.reshape(B, H, S, HD).transpose(2, 0, 1, 3).reshape(N, D)
    h2d, oh = _post_attn(x2d, a2d, out_proj_w, out_proj_b.reshape(1, D),
                         ln1_g.reshape(1, D), ln1_b.reshape(1, D),
                         gate_w, gate_b.reshape(1, E))
    rank, cnt = _ranks(oh)
    dest, te, vd = _schedule(cnt, oh, rank)
    dest1 = dest.reshape(N)
    xs = _sc_scatter(h2d, dest1)
    ys = _ffn(te.reshape(GT), vd.reshape(GT), xs, w1,
              b1.reshape(E, FF // D, 1, D), w2, b2.reshape(E, 1, D))
    ytok = _sc_gather(ys, dest1)
    out = _ln2(h2d, ytok, ln2_g.reshape(1, D), ln2_b.reshape(1, D))
    return out.reshape(S, B, D)


# speedup vs baseline: 4.3584x; 1.1418x over previous
import functools

import jax
import jax.numpy as jnp
from jax import lax
from jax.experimental import pallas as pl
from jax.experimental.pallas import tpu as pltpu
from jax.experimental.pallas import tpu_sc as plsc

S, B, D, H, HD = 2048, 2, 1024, 4, 256
E, FF = 32, 4096
N = S * B
RT = 512
TT = 256
GT = N // TT + E
NP = GT * TT


def _qkv_body(x_ref, w_ref, b_ref, o_ref):
    o_ref[...] = lax.dot_general(
        x_ref[...], w_ref[...], (((1,), (1,)), ((), ())),
        preferred_element_type=jnp.float32) + b_ref[0]


def _qkv(x2d, wi, bi3):
    return pl.pallas_call(
        _qkv_body,
        grid=(N // RT, 3),
        in_specs=[
            pl.BlockSpec((RT, D), lambda i, j: (i, 0)),
            pl.BlockSpec((D, D), lambda i, j: (j, 0)),
            pl.BlockSpec((1, 1, D), lambda i, j: (j, 0, 0)),
        ],
        out_specs=pl.BlockSpec((RT, D), lambda i, j: (i, j)),
        out_shape=jax.ShapeDtypeStruct((N, 3 * D), jnp.float32),
    )(x2d, wi, bi3)


def _attn_body(q_ref, k_ref, v_ref, o_ref):
    q = q_ref[0]
    k = k_ref[0]
    v = v_ref[0]
    s = lax.dot_general(q, k, (((1,), (1,)), ((), ())),
                        preferred_element_type=jnp.float32) * (1.0 / 16.0)
    m = jnp.max(s, axis=-1, keepdims=True)
    p = jnp.exp(s - m)
    p = p / jnp.sum(p, axis=-1, keepdims=True)
    o_ref[0] = jnp.dot(p, v, preferred_element_type=jnp.float32)


def _attn(q, k, v):
    qt = 512
    return pl.pallas_call(
        _attn_body,
        grid=(B * H, S // qt),
        in_specs=[
            pl.BlockSpec((1, qt, HD), lambda h, i: (h, i, 0)),
            pl.BlockSpec((1, S, HD), lambda h, i: (h, 0, 0)),
            pl.BlockSpec((1, S, HD), lambda h, i: (h, 0, 0)),
        ],
        out_specs=pl.BlockSpec((1, qt, HD), lambda h, i: (h, i, 0)),
        out_shape=jax.ShapeDtypeStruct((B * H, S, HD), jnp.float32),
    )(q, k, v)


_NRT = N // RT


def _route_body(x_ref, a_ref, wo_ref, bo_ref, g_ref, b_ref, gw_ref, gb_ref,
                h_ref, dest_ref, te_ref, vd_ref, ohs, cntp, cnt_run):
    i = pl.program_id(0)
    j = lax.rem(i, _NRT)

    @pl.when(i == 0)
    def _():
        cnt_run[...] = jnp.zeros_like(cnt_run)

    @pl.when(i < _NRT)
    def _():
        o = lax.dot_general(a_ref[...], wo_ref[...], (((1,), (1,)), ((), ())),
                            preferred_element_type=jnp.float32) + bo_ref[...]
        s = x_ref[...] + o
        m = jnp.mean(s, axis=-1, keepdims=True)
        c = s - m
        v = jnp.mean(c * c, axis=-1, keepdims=True)
        h = c * lax.rsqrt(v + 1e-5) * g_ref[...] + b_ref[...]
        h_ref[...] = h
        logits = jnp.dot(h, gw_ref[...], preferred_element_type=jnp.float32) \
            + gb_ref[...]
        oh = (logits == jnp.max(logits, axis=-1, keepdims=True)) \
            .astype(jnp.float32)
        strict_up = (lax.broadcasted_iota(jnp.int32, (E, E), 0)
                     < lax.broadcasted_iota(jnp.int32, (E, E), 1)) \
            .astype(jnp.float32)
        earlier = jnp.dot(oh, strict_up, preferred_element_type=jnp.float32)
        oh = oh * (earlier == 0.0).astype(jnp.float32)
        ohs[j] = oh
        cntp[j] = cnt_run[...]
        cnt_run[...] = cnt_run[...] + jnp.sum(oh, axis=0, keepdims=True)

    @pl.when(i >= _NRT)
    def _():
        cnt = cnt_run[...]
        ntiles = jnp.floor((cnt + (TT - 1)) * (1.0 / TT))
        strict_lo = (lax.broadcasted_iota(jnp.int32, (E, E), 0)
                     < lax.broadcasted_iota(jnp.int32, (E, E), 1)) \
            .astype(jnp.float32)
        start = jnp.dot(ntiles, strict_lo,
                        preferred_element_type=jnp.float32)
        pad_off = start * float(TT)
        oh = ohs[j]
        tril = (lax.broadcasted_iota(jnp.int32, (RT, RT), 1)
                < lax.broadcasted_iota(jnp.int32, (RT, RT), 0)) \
            .astype(jnp.float32)
        excl = jnp.dot(tril, oh, preferred_element_type=jnp.float32)
        rank = jnp.sum(oh * (excl + cntp[j]), axis=1, keepdims=True)
        dest = rank + jnp.sum(oh * pad_off, axis=1, keepdims=True)
        dest_ref[...] = dest.astype(jnp.int32)
        total = jnp.sum(ntiles)
        t_iota = lax.broadcasted_iota(jnp.int32, (GT, 1), 0) \
            .astype(jnp.float32)
        t_eff = jnp.minimum(t_iota, total - 1.0)
        te = jnp.sum((t_eff >= start).astype(jnp.float32), axis=1,
                     keepdims=True) - 1.0
        te_ref[...] = te.astype(jnp.int32)
        vd_ref[...] = (t_iota < total).astype(jnp.int32)


def _route(x2d, a2d, wo, bo, g1, b1, gw, gb):
    in_idx = lambda i: (jnp.where(i < _NRT, i, 0), 0)
    const_idx = lambda i: (0, 0)
    return pl.pallas_call(
        _route_body,
        grid=(2 * _NRT,),
        in_specs=[
            pl.BlockSpec((RT, D), in_idx),
            pl.BlockSpec((RT, D), in_idx),
            pl.BlockSpec((D, D), const_idx),
            pl.BlockSpec((1, D), const_idx),
            pl.BlockSpec((1, D), const_idx),
            pl.BlockSpec((1, D), const_idx),
            pl.BlockSpec((D, E), const_idx),
            pl.BlockSpec((1, E), const_idx),
        ],
        out_specs=[
            pl.BlockSpec((RT, D), lambda i: (jnp.where(i < _NRT, i, _NRT - 1),
                                             0)),
            pl.BlockSpec((RT, 1), lambda i: (jnp.where(i < _NRT, 0, i - _NRT),
                                             0)),
            pl.BlockSpec((GT, 1), const_idx),
            pl.BlockSpec((GT, 1), const_idx),
        ],
        out_shape=[
            jax.ShapeDtypeStruct((N, D), jnp.float32),
            jax.ShapeDtypeStruct((N, 1), jnp.int32),
            jax.ShapeDtypeStruct((GT, 1), jnp.int32),
            jax.ShapeDtypeStruct((GT, 1), jnp.int32),
        ],
        scratch_shapes=[
            pltpu.VMEM((_NRT, RT, E), jnp.float32),
            pltpu.VMEM((_NRT, 1, E), jnp.float32),
            pltpu.VMEM((1, E), jnp.float32),
        ],
    )(x2d, a2d, wo, bo, g1, b1, gw, gb)


_NW = 32
_CH = 64


def _sc_mesh():
    return plsc.VectorSubcoreMesh(core_axis_name="c", subcore_axis_name="s")


def _sc_scatter(h2d, dest1):
    @functools.partial(
        pl.kernel, mesh=_sc_mesh(),
        out_type=jax.ShapeDtypeStruct((NP, D), jnp.float32),
        scratch_types=[
            pltpu.VMEM((_CH,), jnp.int32),
            pltpu.VMEM((_CH, D), jnp.float32),
            pltpu.SemaphoreType.DMA,
        ],
    )
    def body(h_hbm, dest_hbm, xs_hbm, idx_v, rows_v, sem):
        wid = lax.axis_index("s") * 2 + lax.axis_index("c")
        base = wid * (N // _NW)
        for c in range(N // _NW // _CH):
            off = base + c * _CH
            pltpu.sync_copy(dest_hbm.at[pl.ds(off, _CH)], idx_v)
            pltpu.sync_copy(h_hbm.at[pl.ds(off, _CH)], rows_v)
            pltpu.async_copy(rows_v, xs_hbm.at[idx_v], sem).wait()

    return body(h2d, dest1)


def _ffn_body(te_ref, vd_ref, x_ref, w1_ref, b1_ref, w2_ref, b2_ref, y_ref,
              acc):
    t = pl.program_id(0)
    f = pl.program_id(1)

    @pl.when(vd_ref[t] == 1)
    def _():
        h1 = jax.nn.relu(
            jnp.dot(x_ref[...], w1_ref[0], preferred_element_type=jnp.float32)
            + b1_ref[0, 0])
        part = jnp.dot(h1, w2_ref[0], preferred_element_type=jnp.float32)

        @pl.when(f == 0)
        def _():
            acc[...] = part

        @pl.when(f > 0)
        def _():
            acc[...] = acc[...] + part

        @pl.when(f == FF // D - 1)
        def _():
            y_ref[...] = acc[...] + b2_ref[0]


def _ffn(te, vd, xs, w1, b1, w2, b2):
    nf = FF // D
    grid_spec = pltpu.PrefetchScalarGridSpec(
        num_scalar_prefetch=2,
        grid=(GT, nf),
        in_specs=[
            pl.BlockSpec((TT, D), lambda t, f, te, vd: (t * vd[t], 0)),
            pl.BlockSpec((1, D, D), lambda t, f, te, vd: (te[t], 0, f * vd[t])),
            pl.BlockSpec((1, 1, 1, D),
                         lambda t, f, te, vd: (te[t], f * vd[t], 0, 0)),
            pl.BlockSpec((1, D, D), lambda t, f, te, vd: (te[t], f * vd[t], 0)),
            pl.BlockSpec((1, 1, D), lambda t, f, te, vd: (te[t], 0, 0)),
        ],
        out_specs=pl.BlockSpec((TT, D), lambda t, f, te, vd: (t, 0)),
        scratch_shapes=[pltpu.VMEM((TT, D), jnp.float32)],
    )
    return pl.pallas_call(
        _ffn_body,
        grid_spec=grid_spec,
        out_shape=jax.ShapeDtypeStruct((NP, D), jnp.float32),
    )(te, vd, xs, w1, b1, w2, b2)


def _sc_gather(ys, dest1):
    @functools.partial(
        pl.kernel, mesh=_sc_mesh(),
        out_type=jax.ShapeDtypeStruct((N, D), jnp.float32),
        scratch_types=[
            pltpu.VMEM((_CH,), jnp.int32),
            pltpu.VMEM((_CH, D), jnp.float32),
            pltpu.SemaphoreType.DMA,
        ],
    )
    def body(ys_hbm, dest_hbm, out_hbm, idx_v, rows_v, sem):
        wid = lax.axis_index("s") * 2 + lax.axis_index("c")
        base = wid * (N // _NW)
        for c in range(N // _NW // _CH):
            off = base + c * _CH
            pltpu.sync_copy(dest_hbm.at[pl.ds(off, _CH)], idx_v)
            pltpu.async_copy(ys_hbm.at[idx_v], rows_v, sem).wait()
            pltpu.sync_copy(rows_v, out_hbm.at[pl.ds(off, _CH)])

    return body(ys, dest1)


def _ln2_body(h_ref, y_ref, g_ref, b_ref, o_ref):
    s = h_ref[...] + y_ref[...]
    m = jnp.mean(s, axis=-1, keepdims=True)
    c = s - m
    v = jnp.mean(c * c, axis=-1, keepdims=True)
    o_ref[...] = c * lax.rsqrt(v + 1e-5) * g_ref[...] + b_ref[...]


def _ln2(h2d, ytok, g2, b2):
    return pl.pallas_call(
        _ln2_body,
        grid=(N // RT,),
        in_specs=[
            pl.BlockSpec((RT, D), lambda i: (i, 0)),
            pl.BlockSpec((RT, D), lambda i: (i, 0)),
            pl.BlockSpec((1, D), lambda i: (0, 0)),
            pl.BlockSpec((1, D), lambda i: (0, 0)),
        ],
        out_specs=pl.BlockSpec((RT, D), lambda i: (i, 0)),
        out_shape=jax.ShapeDtypeStruct((N, D), jnp.float32),
    )(h2d, ytok, g2, b2)


def kernel(x, in_proj_w, in_proj_b, out_proj_w, out_proj_b, ln1_g, ln1_b,
           ln2_g, ln2_b, gate_w, gate_b, w1, b1, w2, b2):
    x2d = x.reshape(N, D)
    qkv = _qkv(x2d, in_proj_w, in_proj_b.reshape(3, 1, D))
    qkv_t = qkv.reshape(S, B, 3, H, HD).transpose(2, 1, 3, 0, 4) \
        .reshape(3, B * H, S, HD)
    o = _attn(qkv_t[0], qkv_t[1], qkv_t[2])
    a2d = o.reshape(B, H, S, HD).transpose(2, 0, 1, 3).reshape(N, D)
    h2d, dest, te, vd = _route(x2d, a2d, out_proj_w, out_proj_b.reshape(1, D),
                               ln1_g.reshape(1, D), ln1_b.reshape(1, D),
                               gate_w, gate_b.reshape(1, E))
    dest1 = dest.reshape(N)
    xs = _sc_scatter(h2d, dest1)
    ys = _ffn(te.reshape(GT), vd.reshape(GT), xs, w1,
              b1.reshape(E, FF // D, 1, D), w2, b2.reshape(E, 1, D))
    ytok = _sc_gather(ys, dest1)
    out = _ln2(h2d, ytok, ln2_g.reshape(1, D), ln2_b.reshape(1, D))
    return out.reshape(S, B, D)
